# baseline (speedup 1.0000x reference)
import functools

import jax
import jax.numpy as jnp
from jax import lax
from jax.experimental import pallas as pl
from jax.experimental.pallas import tpu as pltpu
from jax.experimental.pallas import tpu_sc as plsc

_ROWS = 16384
_COLS = 8192
_NIDX = 128
_NC, _NS = 2, 16
_NW = _NC * _NS
_RPW = _ROWS // _NW
_CR = 4
_NCH = _RPW // _CR
_CW = _CR * _COLS
_FR = 64
_FW = _FR * _NIDX
_CPF = _FR // _CR
_LANES = 16


def _make_sc_kernel():
    mesh = plsc.VectorSubcoreMesh(core_axis_name="c", subcore_axis_name="s")

    @functools.partial(
        pl.kernel,
        mesh=mesh,
        out_type=jax.ShapeDtypeStruct((_ROWS * _NIDX,), jnp.float32),
        scratch_types=[
            pltpu.VMEM((_NIDX,), jnp.int32),
            pltpu.VMEM((_CR * _NIDX,), jnp.int32),
            pltpu.VMEM((2 * _CW,), jnp.float32),
            pltpu.VMEM((2 * _FW,), jnp.float32),
            pltpu.SemaphoreType.DMA((2,)),
            pltpu.SemaphoreType.DMA((2,)),
        ],
        compiler_params=pltpu.CompilerParams(
            use_tc_tiling_on_sc=False, needs_layout_passes=False
        ),
    )
    def k(x_hbm, idx_hbm, out_hbm, idx0, pidx, ibuf, obuf, isems, osems):
        wid = lax.axis_index("s") * _NC + lax.axis_index("c")
        r0 = wid * _RPW
        pltpu.sync_copy(idx_hbm, idx0)

        nvec = _NIDX // _LANES

        for rr in range(_CR):
            for kk in range(nvec):
                v = idx0[pl.ds(kk * _LANES, _LANES)] + rr * _COLS
                pidx[pl.ds(rr * _NIDX + kk * _LANES, _LANES)] = v

        def in_copy(c, p):
            return pltpu.make_async_copy(
                x_hbm.at[pl.ds((r0 + c * _CR) * _COLS, _CW)],
                ibuf.at[pl.ds(p * _CW, _CW)],
                isems.at[p],
            )

        def out_copy(f, q):
            return pltpu.make_async_copy(
                obuf.at[pl.ds(q * _FW, _FW)],
                out_hbm.at[pl.ds(r0 * _NIDX + f * _FW, _FW)],
                osems.at[q],
            )

        in_copy(0, 0).start()
        in_copy(1, 1).start()

        def body(t, carry):
            p = lax.rem(t, 2)
            f = lax.div(t, _CPF)
            q = lax.rem(f, 2)
            tin = lax.rem(t, _CPF)

            @pl.when(jnp.logical_and(tin == 0, t >= 2 * _CPF))
            def _():
                out_copy(0, q).wait()

            in_copy(t, p).wait()

            ibase = p * _CW
            obase = q * _FW + tin * _CR * _NIDX
            for rr in range(_CR):
                for kk in range(nvec):
                    iv = pidx[pl.ds(rr * _NIDX + kk * _LANES, _LANES)] + ibase
                    v = plsc.load_gather(ibuf, [iv])
                    obuf[pl.ds(obase + rr * _NIDX + kk * _LANES, _LANES)] = v

            @pl.when(t + 2 < _NCH)
            def _():
                in_copy(t + 2, p).start()

            @pl.when(tin == _CPF - 1)
            def _():
                out_copy(f, q).start()

            return carry

        lax.fori_loop(0, _NCH, body, 0)
        out_copy(0, 0).wait()
        out_copy(0, 1).wait()

    return k


def kernel(x, indices):
    out = _make_sc_kernel()(x.reshape(_ROWS * _COLS), indices)
    return out.reshape(_ROWS, _NIDX)


# speedup vs baseline: 1.3310x; 1.3310x over previous
import functools

import jax
import jax.numpy as jnp
from jax import lax
from jax.experimental import pallas as pl
from jax.experimental.pallas import tpu as pltpu
from jax.experimental.pallas import tpu_sc as plsc

_ROWS = 16384
_COLS = 8192
_NIDX = 128
_NC, _NS = 2, 16
_NW = _NC * _NS
_RPW = _ROWS // _NW
_CPD = 4
_UNROLL = 8
_LANES = 16
_IPD = _CPD * _NIDX
_NCHUNK = _RPW // _CPD


def _make_sc_kernel():
    mesh = plsc.VectorSubcoreMesh(core_axis_name="c", subcore_axis_name="s")

    @functools.partial(
        pl.kernel,
        mesh=mesh,
        out_type=jax.ShapeDtypeStruct((_ROWS * _NIDX,), jnp.float32),
        scratch_types=[
            pltpu.VMEM((_NIDX,), jnp.int32),
            pltpu.VMEM((2 * _UNROLL * _IPD,), jnp.int32),
            pltpu.VMEM((_RPW * _NIDX,), jnp.float32),
            pltpu.SemaphoreType.DMA((2,)),
        ],
        compiler_params=pltpu.CompilerParams(use_tc_tiling_on_sc=False),
    )
    def k(x_hbm, idx_hbm, out_hbm, idx0, idxb, obuf, sems):
        wid = lax.axis_index("s") * _NC + lax.axis_index("c")
        r0 = wid * _RPW
        pltpu.sync_copy(idx_hbm, idx0)

        nvec = _NIDX // _LANES

        def init_body(s, carry):
            for rr in range(_CPD):
                base = (r0 + s * _CPD + rr) * _COLS
                for kk in range(nvec):
                    v = idx0[pl.ds(kk * _LANES, _LANES)] + base
                    off = s * _IPD + rr * _NIDX + kk * _LANES
                    idxb[pl.ds(off, _LANES)] = v
            return carry

        lax.fori_loop(0, 2 * _UNROLL, init_body, 0)

        n_iters = _NCHUNK // _UNROLL

        def start_group(t):
            p = lax.rem(t, 2)
            for u in range(_UNROLL):
                c = t * _UNROLL + u
                s = p * _UNROLL + u
                pltpu.make_async_copy(
                    x_hbm.at[idxb.at[pl.ds(s * _IPD, _IPD)]],
                    obuf.at[pl.ds(c * _IPD, _IPD)],
                    sems.at[p],
                ).start()

        def drain_group(t):
            p = lax.rem(t, 2)
            for _ in range(_UNROLL):
                pltpu.make_async_copy(
                    x_hbm.at[idxb.at[pl.ds(0, _IPD)]],
                    obuf.at[pl.ds(0, _IPD)],
                    sems.at[p],
                ).wait()

        def bump_group(t):
            p = lax.rem(t, 2)
            step = 2 * _UNROLL * _CPD * _COLS
            for u in range(_UNROLL):
                s = p * _UNROLL + u
                for kk in range(_IPD // _LANES):
                    off = s * _IPD + kk * _LANES
                    idxb[pl.ds(off, _LANES)] = idxb[pl.ds(off, _LANES)] + step

        def body(t, carry):
            start_group(t)

            @pl.when(t > 0)
            def _():
                drain_group(t - 1)
                bump_group(t - 1)

            return carry

        lax.fori_loop(0, n_iters, body, 0)
        drain_group(n_iters - 1)
        pltpu.sync_copy(obuf, out_hbm.at[pl.ds(r0 * _NIDX, _RPW * _NIDX)])

    return k


def kernel(x, indices):
    out = _make_sc_kernel()(x.reshape(_ROWS * _COLS), indices)
    return out.reshape(_ROWS, _NIDX)


# speedup vs baseline: 3.7817x; 2.8411x over previous
import jax
import jax.numpy as jnp
from jax.experimental import pallas as pl
from jax.experimental.pallas import tpu as pltpu

_ROWS = 16384
_COLS = 8192
_NIDX = 128
_RB = 256


def _body(x_ref, s_ref, o_ref):
    o_ref[...] = jax.lax.dot_general(
        x_ref[...],
        s_ref[...],
        (((1,), (0,)), ((), ())),
        preferred_element_type=jnp.float32,
    )


def kernel(x, indices):
    onehot = (
        jax.lax.broadcasted_iota(jnp.int32, (_COLS, _NIDX), 0)
        == indices[None, :]
    ).astype(jnp.float32)
    return pl.pallas_call(
        _body,
        grid=(_ROWS // _RB,),
        in_specs=[
            pl.BlockSpec((_RB, _COLS), lambda i: (i, 0)),
            pl.BlockSpec((_COLS, _NIDX), lambda i: (0, 0)),
        ],
        out_specs=pl.BlockSpec((_RB, _NIDX), lambda i: (i, 0)),
        out_shape=jax.ShapeDtypeStruct((_ROWS, _NIDX), jnp.float32),
        compiler_params=pltpu.CompilerParams(
            dimension_semantics=("arbitrary",),
        ),
    )(x, onehot)
